# Initial kernel scaffold; baseline (speedup 1.0000x reference)
import jax
import jax.numpy as jnp
from jax import lax
from jax.experimental import pallas as pl
from jax.experimental.pallas import tpu as pltpu
from jax.experimental.pallas import tpu_sc as plsc

D_MODEL = 64
NUM_WORKERS = 32
CHUNK = 128


def _emb_body(x_hbm, table_hbm, out_hbm, idx_v, rows_v, sem):
    wid = lax.axis_index("s") * 2 + lax.axis_index("c")
    b_total = x_hbm.shape[0]
    b_per_w = b_total // NUM_WORKERS
    n_chunks = b_per_w // CHUNK
    base = wid * b_per_w

    def body(i, carry):
        off = base + i * CHUNK
        pltpu.sync_copy(x_hbm.at[pl.ds(off, CHUNK)], idx_v)
        pltpu.async_copy(table_hbm.at[idx_v], rows_v, sem).wait()
        pltpu.sync_copy(rows_v, out_hbm.at[pl.ds(off, CHUNK)])
        return carry

    lax.fori_loop(0, n_chunks, body, 0)


@jax.jit
def kernel(x, table):
    b, h = x.shape
    xf = x.reshape(-1).astype(jnp.int32)
    mesh = plsc.VectorSubcoreMesh(core_axis_name="c", subcore_axis_name="s")
    gather = pl.kernel(
        _emb_body,
        out_type=jax.ShapeDtypeStruct((b * h, D_MODEL), jnp.float32),
        mesh=mesh,
        scratch_types=[
            pltpu.VMEM((CHUNK,), jnp.int32),
            pltpu.VMEM((CHUNK, D_MODEL), jnp.float32),
            pltpu.SemaphoreType.DMA,
        ],
    )
    out = gather(xf, table)
    return out.reshape(b, h, D_MODEL)


# speedup vs baseline: 1.5732x; 1.5732x over previous
import jax
import jax.numpy as jnp
from jax import lax
from jax.experimental import pallas as pl
from jax.experimental.pallas import tpu as pltpu
from jax.experimental.pallas import tpu_sc as plsc

D_MODEL = 64
NUM_WORKERS = 32
CHUNK = 128


def _emb_body(x_hbm, table_hbm, out_hbm, idx_v, rows_v, sem):
    wid = lax.axis_index("s") * 2 + lax.axis_index("c")
    b_total = x_hbm.shape[0]
    b_per_w = b_total // NUM_WORKERS
    n_chunks = b_per_w // CHUNK
    base = wid * b_per_w

    def body(i, carry):
        off = base + i * CHUNK
        pltpu.sync_copy(x_hbm.at[pl.ds(off, CHUNK)], idx_v)
        pltpu.async_copy(table_hbm.at[idx_v], rows_v, sem).wait()
        pltpu.sync_copy(rows_v, out_hbm.at[pl.ds(off, CHUNK)])
        return carry

    lax.fori_loop(0, n_chunks, body, 0)


@jax.jit
def kernel(x, table):
    b, h = x.shape
    xf = x.reshape(-1).astype(jnp.int32)
    mesh = plsc.VectorSubcoreMesh(core_axis_name="c", subcore_axis_name="s")
    gather = pl.kernel(
        _emb_body,
        out_type=jax.ShapeDtypeStruct((b * h, D_MODEL), jnp.float32),
        mesh=mesh,
        scratch_types=[
            pltpu.VMEM((CHUNK,), jnp.int32),
            pltpu.VMEM((CHUNK, D_MODEL), jnp.float32),
            pltpu.SemaphoreType.DMA,
        ],
        compiler_params=pltpu.CompilerParams(use_tc_tiling_on_sc=False),
    )
    out = gather(xf, table)
    return out.reshape(b, h, D_MODEL)


# speedup vs baseline: 1.8642x; 1.1850x over previous
import jax
import jax.numpy as jnp
from jax import lax
from jax.experimental import pallas as pl
from jax.experimental.pallas import tpu as pltpu
from jax.experimental.pallas import tpu_sc as plsc

D_MODEL = 64
NUM_WORKERS = 32
CHUNK = 128
GATHERS_PER_SUPER = 4
SUPER = CHUNK * GATHERS_PER_SUPER


def _emb_body(x_hbm, table_hbm, out_hbm, idx_v, rows_a, rows_b, gsem_a,
              gsem_b, ssem_a, ssem_b):
    wid = lax.axis_index("s") * 2 + lax.axis_index("c")
    n_chunks_total = x_hbm.shape[0]
    chunks_per_w = n_chunks_total // NUM_WORKERS
    n_super = chunks_per_w // GATHERS_PER_SUPER
    chunk0 = wid * chunks_per_w
    row0 = chunk0 * CHUNK

    pltpu.sync_copy(x_hbm.at[pl.ds(chunk0, chunks_per_w)], idx_v)

    def start_gathers(buf, sem, s):
        for b in range(GATHERS_PER_SUPER):
            c = s * GATHERS_PER_SUPER + b
            pltpu.async_copy(
                table_hbm.at[idx_v.at[c]],
                buf.at[pl.ds(b * CHUNK, CHUNK)],
                sem,
            )

    def wait_gathers(buf, sem):
        pltpu.make_async_copy(table_hbm.at[pl.ds(0, SUPER)], buf, sem).wait()

    def start_store(buf, sem, s):
        pltpu.async_copy(buf, out_hbm.at[pl.ds(row0 + s * SUPER, SUPER)], sem)

    def wait_store(buf, sem, s):
        pltpu.make_async_copy(
            buf, out_hbm.at[pl.ds(row0 + s * SUPER, SUPER)], sem).wait()

    start_gathers(rows_a, gsem_a, 0)

    def body(i, carry):
        sa = 2 * i
        sb = 2 * i + 1

        @pl.when(i > 0)
        def _():
            wait_store(rows_b, ssem_b, sb - 2)
        start_gathers(rows_b, gsem_b, sb)

        wait_gathers(rows_a, gsem_a)
        start_store(rows_a, ssem_a, sa)

        wait_gathers(rows_b, gsem_b)
        start_store(rows_b, ssem_b, sb)

        @pl.when(i < n_super // 2 - 1)
        def _():
            wait_store(rows_a, ssem_a, sa)
            start_gathers(rows_a, gsem_a, sa + 2)

        return carry

    lax.fori_loop(0, n_super // 2, body, 0)

    wait_store(rows_a, ssem_a, n_super - 2)
    wait_store(rows_b, ssem_b, n_super - 1)


@jax.jit
def kernel(x, table):
    b, h = x.shape
    n_rows = b * h
    xf = x.reshape(n_rows // CHUNK, CHUNK).astype(jnp.int32)
    chunks_per_w = (n_rows // CHUNK) // NUM_WORKERS
    mesh = plsc.VectorSubcoreMesh(core_axis_name="c", subcore_axis_name="s")
    gather = pl.kernel(
        _emb_body,
        out_type=jax.ShapeDtypeStruct((n_rows, D_MODEL), jnp.float32),
        mesh=mesh,
        scratch_types=[
            pltpu.VMEM((chunks_per_w, CHUNK), jnp.int32),
            pltpu.VMEM((SUPER, D_MODEL), jnp.float32),
            pltpu.VMEM((SUPER, D_MODEL), jnp.float32),
            pltpu.SemaphoreType.DMA,
            pltpu.SemaphoreType.DMA,
            pltpu.SemaphoreType.DMA,
            pltpu.SemaphoreType.DMA,
        ],
        compiler_params=pltpu.CompilerParams(use_tc_tiling_on_sc=False),
    )
    out = gather(xf, table)
    return out.reshape(b, h, D_MODEL)
